# baseline (speedup 1.0000x reference)
import jax
import jax.numpy as jnp
from jax.experimental import pallas as pl
from jax.experimental.pallas import tpu as pltpu


def _body(x_hbm, a_hbm, c_ref, w1_ref, b1_ref, cw_ref, o_ref,
          a_vmem, x_vmem, xall, abf, d_vmem, wbig, sem_x, sem_a):
    B, N, D = x_hbm.shape
    K = cw_ref.shape[0]
    OUT = cw_ref.shape[3]

    hb = B // 2
    hn = N // 2
    cp_x = [
        pltpu.make_async_copy(x_hbm.at[pl.ds(i * hb, hb)],
                              x_vmem.at[pl.ds(i * hb, hb)], sem_x.at[i])
        for i in range(2)
    ]
    cp_a = [
        pltpu.make_async_copy(a_hbm.at[pl.ds(i * hn, hn), :],
                              a_vmem.at[pl.ds(i * hn, hn), :], sem_a.at[i])
        for i in range(2)
    ]
    cp_x[0].start()
    cp_x[1].start()
    cp_a[0].start()
    cp_a[1].start()

    wbig[:, :] = jnp.zeros((B * D, K * 128), jnp.bfloat16)
    for k in range(K):
        for b in range(B):
            wbig[D * b:D * (b + 1),
                 128 * k + OUT * b:128 * k + OUT * b + OUT] = (
                cw_ref[k, 0].astype(jnp.bfloat16))

    for b in range(B):
        if b % hb == 0:
            cp_x[b // hb].wait()
        cb = c_ref[b, 0:1, :]
        sc = cb * jax.nn.sigmoid(cb)
        mod = jnp.dot(sc, w1_ref[:, :], preferred_element_type=jnp.float32)
        mod = mod + b1_ref[0:1, :]
        shift = mod[:, :D]
        scale = mod[:, D:]
        xb = x_vmem[b]
        mu = jnp.mean(xb, axis=1, keepdims=True)
        xc = xb - mu
        var = jnp.mean(xc * xc, axis=1, keepdims=True)
        xn = xc * jax.lax.rsqrt(var + 1e-6)
        xm = xn * (1.0 + scale) + shift
        xall[:, D * b:D * (b + 1)] = xm.astype(jnp.bfloat16)

    zall = jnp.dot(xall[:, :], wbig[:, :], preferred_element_type=jnp.float32)
    z0 = zall[:, 0:128]
    z1 = zall[:, 128:256]
    z2 = zall[:, 256:384]

    for i in range(2):
        cp_a[i].wait()
        rows = a_vmem[pl.ds(i * hn, hn), :]
        d_vmem[pl.ds(i * hn, hn), :] = jnp.sum(rows, axis=1, keepdims=True)
        abf[pl.ds(i * hn, hn), :] = rows.astype(jnp.bfloat16)

    s = jax.lax.rsqrt(d_vmem[:, :])
    a = abf[:, :]

    def lap(y):
        u = jnp.dot(a, (s * y).astype(jnp.bfloat16),
                    preferred_element_type=jnp.float32)
        return y - s * u

    t = lap(z2)
    w = lap(z1 + 2.0 * t)
    o_ref[:, :] = z0 - z2 + w


def kernel(x, adj, c, W1, b1, cheb_w, cheb_b):
    B, N, D = x.shape
    K, _, _, OUT = cheb_w.shape

    b1r = b1.reshape(1, 2 * D)

    out_full = pl.pallas_call(
        _body,
        out_shape=jax.ShapeDtypeStruct((N, 128), jnp.float32),
        in_specs=[
            pl.BlockSpec(memory_space=pl.ANY),
            pl.BlockSpec(memory_space=pl.ANY),
            pl.BlockSpec(memory_space=pltpu.VMEM),
            pl.BlockSpec(memory_space=pltpu.VMEM),
            pl.BlockSpec(memory_space=pltpu.VMEM),
            pl.BlockSpec(memory_space=pltpu.VMEM),
        ],
        scratch_shapes=[
            pltpu.VMEM((N, N), jnp.float32),
            pltpu.VMEM((B, N, D), jnp.float32),
            pltpu.VMEM((N, B * D), jnp.bfloat16),
            pltpu.VMEM((N, N), jnp.bfloat16),
            pltpu.VMEM((N, 1), jnp.float32),
            pltpu.VMEM((B * D, K * 128), jnp.bfloat16),
            pltpu.SemaphoreType.DMA((2,)),
            pltpu.SemaphoreType.DMA((2,)),
        ],
        compiler_params=pltpu.CompilerParams(
            vmem_limit_bytes=100 * 1024 * 1024,
        ),
    )(x, adj, c, W1, b1r, cheb_w)

    out = out_full[:, :B * OUT].reshape(N, B, OUT).transpose(1, 0, 2) + cheb_b
    return out


# speedup vs baseline: 1.0486x; 1.0486x over previous
import jax
import jax.numpy as jnp
from jax.experimental import pallas as pl
from jax.experimental.pallas import tpu as pltpu


def _body(x_hbm, a_hbm, c_ref, w1_ref, b1_ref, cw_ref, o_ref,
          a_vmem, x_vmem, xall, d_vmem, wbig, sem_x, sem_a):
    B, N, D = x_hbm.shape
    K = cw_ref.shape[0]
    OUT = cw_ref.shape[3]

    cp_x = pltpu.make_async_copy(x_hbm, x_vmem, sem_x)
    cp_a = pltpu.make_async_copy(a_hbm, a_vmem, sem_a)
    cp_x.start()
    cp_a.start()

    wbig[:, :] = jnp.zeros((B * D, K * 128), jnp.float32)
    for k in range(K):
        for b in range(B):
            wbig[D * b:D * (b + 1),
                 128 * k + OUT * b:128 * k + OUT * b + OUT] = cw_ref[k, 0]

    cp_x.wait()
    for b in range(B):
        cb = c_ref[b, 0:1, :]
        sc = cb * jax.nn.sigmoid(cb)
        mod = jnp.dot(sc, w1_ref[:, :], preferred_element_type=jnp.float32)
        mod = mod + b1_ref[0:1, :]
        shift = mod[:, :D]
        scale = mod[:, D:]
        xb = x_vmem[b]
        mu = jnp.mean(xb, axis=1, keepdims=True)
        xc = xb - mu
        var = jnp.mean(xc * xc, axis=1, keepdims=True)
        xn = xc * jax.lax.rsqrt(var + 1e-6)
        xm = xn * (1.0 + scale) + shift
        xall[:, D * b:D * (b + 1)] = xm

    zall = jnp.dot(xall[:, :], wbig[:, :], preferred_element_type=jnp.float32)
    z0 = zall[:, 0:128]
    z1 = zall[:, 128:256]
    z2 = zall[:, 256:384]

    cp_a.wait()
    a = a_vmem[:, :]
    d_vmem[:, :] = jnp.sum(a, axis=1, keepdims=True)
    s = jax.lax.rsqrt(d_vmem[:, :])

    def lap(y):
        u = jnp.dot(a, s * y, preferred_element_type=jnp.float32)
        return y - s * u

    t = lap(z2)
    w = lap(z1 + 2.0 * t)
    o_ref[:, :] = z0 - z2 + w


def kernel(x, adj, c, W1, b1, cheb_w, cheb_b):
    B, N, D = x.shape
    K, _, _, OUT = cheb_w.shape

    b1r = b1.reshape(1, 2 * D)

    out_full = pl.pallas_call(
        _body,
        out_shape=jax.ShapeDtypeStruct((N, 128), jnp.float32),
        in_specs=[
            pl.BlockSpec(memory_space=pl.ANY),
            pl.BlockSpec(memory_space=pl.ANY),
            pl.BlockSpec(memory_space=pltpu.VMEM),
            pl.BlockSpec(memory_space=pltpu.VMEM),
            pl.BlockSpec(memory_space=pltpu.VMEM),
            pl.BlockSpec(memory_space=pltpu.VMEM),
        ],
        scratch_shapes=[
            pltpu.VMEM((N, N), jnp.float32),
            pltpu.VMEM((B, N, D), jnp.float32),
            pltpu.VMEM((N, B * D), jnp.float32),
            pltpu.VMEM((N, 1), jnp.float32),
            pltpu.VMEM((B * D, K * 128), jnp.float32),
            pltpu.SemaphoreType.DMA,
            pltpu.SemaphoreType.DMA,
        ],
        compiler_params=pltpu.CompilerParams(
            vmem_limit_bytes=100 * 1024 * 1024,
        ),
    )(x, adj, c, W1, b1r, cheb_w)

    out = out_full[:, :B * OUT].reshape(N, B, OUT).transpose(1, 0, 2) + cheb_b
    return out


# speedup vs baseline: 1.0503x; 1.0016x over previous
import jax
import jax.numpy as jnp
from jax.experimental import pallas as pl
from jax.experimental.pallas import tpu as pltpu


def _body(x_hbm, a_hbm, c_ref, w1_ref, b1_ref, cw_ref, o_ref,
          a_vmem, x_vmem, xall, d_vmem, wbig, sem_x, sem_a):
    B, N, D = x_hbm.shape
    K = cw_ref.shape[0]
    OUT = cw_ref.shape[3]

    cp_x = pltpu.make_async_copy(x_hbm, x_vmem, sem_x)
    cp_a = pltpu.make_async_copy(a_hbm, a_vmem, sem_a)
    cp_x.start()
    cp_a.start()

    wbig[:, :] = jnp.zeros((B * D, K * 128), jnp.float32)
    for k in range(K):
        for b in range(B):
            wbig[D * b:D * (b + 1),
                 128 * k + OUT * b:128 * k + OUT * b + OUT] = cw_ref[k, 0]

    cp_x.wait()
    for b in range(B):
        cb = c_ref[b, 0:1, :]
        sc = cb * jax.nn.sigmoid(cb)
        mod = jnp.dot(sc, w1_ref[:, :], preferred_element_type=jnp.float32)
        mod = mod + b1_ref[0:1, :]
        shift = mod[:, :D]
        scale = mod[:, D:]
        xb = x_vmem[b]
        mu = jnp.mean(xb, axis=1, keepdims=True)
        xc = xb - mu
        var = jnp.mean(xc * xc, axis=1, keepdims=True)
        xn = xc * jax.lax.rsqrt(var + 1e-6)
        xm = xn * (1.0 + scale) + shift
        xall[:, D * b:D * (b + 1)] = xm

    zall = jnp.dot(xall[:, :], wbig[:, :], preferred_element_type=jnp.float32)
    z0 = zall[:, 0:128]
    z1 = zall[:, 128:256]
    z2 = zall[:, 256:384]

    cp_a.wait()
    a = a_vmem[:, :]
    d_vmem[:, :] = jnp.sum(a, axis=1, keepdims=True)
    s = jax.lax.rsqrt(d_vmem[:, :])

    def lap(y):
        u = jnp.dot(a, s * y, precision=jax.lax.Precision.DEFAULT,
                    preferred_element_type=jnp.float32)
        return y - s * u

    t = lap(z2)
    w = lap(z1 + 2.0 * t)
    o_ref[:, :] = z0 - z2 + w


def kernel(x, adj, c, W1, b1, cheb_w, cheb_b):
    B, N, D = x.shape
    K, _, _, OUT = cheb_w.shape

    b1r = b1.reshape(1, 2 * D)

    out_full = pl.pallas_call(
        _body,
        out_shape=jax.ShapeDtypeStruct((N, 128), jnp.float32),
        in_specs=[
            pl.BlockSpec(memory_space=pl.ANY),
            pl.BlockSpec(memory_space=pl.ANY),
            pl.BlockSpec(memory_space=pltpu.VMEM),
            pl.BlockSpec(memory_space=pltpu.VMEM),
            pl.BlockSpec(memory_space=pltpu.VMEM),
            pl.BlockSpec(memory_space=pltpu.VMEM),
        ],
        scratch_shapes=[
            pltpu.VMEM((N, N), jnp.float32),
            pltpu.VMEM((B, N, D), jnp.float32),
            pltpu.VMEM((N, B * D), jnp.float32),
            pltpu.VMEM((N, 1), jnp.float32),
            pltpu.VMEM((B * D, K * 128), jnp.float32),
            pltpu.SemaphoreType.DMA,
            pltpu.SemaphoreType.DMA,
        ],
        compiler_params=pltpu.CompilerParams(
            vmem_limit_bytes=100 * 1024 * 1024,
        ),
    )(x, adj, c, W1, b1r, cheb_w)

    out = out_full[:, :B * OUT].reshape(N, B, OUT).transpose(1, 0, 2) + cheb_b
    return out
